# baseline (speedup 1.0000x reference)
import functools

import jax
import jax.numpy as jnp
from jax import lax
from jax.experimental import pallas as pl
from jax.experimental.pallas import tpu as pltpu
from jax.experimental.pallas import tpu_sc as plsc

L = 16
TS, TL = 8, 128
DEPTH = 8


def kernel(u_idx, i_idx, u_emb, i_emb, u_bias, i_bias):
    B = u_idx.shape[0]
    N, F = u_emb.shape
    info = plsc.get_sparse_core_info()
    NC, NS = info.num_cores, info.num_subcores
    NW = NC * NS
    b_per_w = B // NW

    n_tc = (N + TL - 1) // TL
    n_tiles = (F // TS) * n_tc
    tpw = (n_tiles + NW - 1) // NW
    flat_len = n_tiles * TS * TL

    mesh = plsc.VectorSubcoreMesh(core_axis_name="c", subcore_axis_name="s")

    @functools.partial(
        pl.kernel,
        mesh=mesh,
        out_type=(jax.ShapeDtypeStruct((n_tiles * TS, TL), jnp.float32),
                  jax.ShapeDtypeStruct((n_tiles * TS, TL), jnp.float32)),
        compiler_params=pltpu.CompilerParams(use_tc_tiling_on_sc=True),
        scratch_types=[pltpu.SemaphoreType.DMA],
    )
    def detile(ut_hbm, it_hbm, utp_hbm, itp_hbm, cu_hbm, ci_hbm, sem):
        wid = lax.axis_index("s") * NC + lax.axis_index("c")
        t0 = wid * tpw

        def fire(t, src_full, src_tail, dst):
            a = t // n_tc
            c = t % n_tc
            a8 = pl.multiple_of(a * TS, TS)
            d8 = pl.multiple_of(t * TS, TS)

            @pl.when(c != n_tc - 1)
            def _():
                cl = pl.multiple_of(c * TL, TL)
                pltpu.async_copy(src_full.at[pl.ds(a8, TS), pl.ds(cl, TL)],
                                 dst.at[pl.ds(d8, TS)], sem)

            @pl.when(c == n_tc - 1)
            def _():
                pltpu.async_copy(src_tail.at[pl.ds(a8, TS), :],
                                 dst.at[pl.ds(d8, TS)], sem)

        def drain(t, dst):
            d8 = pl.multiple_of(t * TS, TS)
            pltpu.make_async_copy(ut_hbm.at[pl.ds(0, TS), pl.ds(0, TL)],
                                  dst.at[pl.ds(d8, TS)], sem).wait()

        def body(k, carry):
            t = t0 + k

            @pl.when(t < n_tiles)
            def _():
                fire(t, ut_hbm, utp_hbm, cu_hbm)
                fire(t, it_hbm, itp_hbm, ci_hbm)

            t8 = t - DEPTH

            @pl.when((k >= DEPTH) & (t8 < n_tiles))
            def _():
                drain(t8, cu_hbm)
                drain(t8, ci_hbm)

            return carry

        lax.fori_loop(0, tpw, body, 0)

        def tail(k, carry):
            t8 = t0 + tpw - DEPTH + k

            @pl.when(t8 < n_tiles)
            def _():
                drain(t8, cu_hbm)
                drain(t8, ci_hbm)

            return carry

        lax.fori_loop(0, DEPTH, tail, 0)

    @functools.partial(
        pl.kernel,
        mesh=mesh,
        out_type=jax.ShapeDtypeStruct((B,), jnp.float32),
        compiler_params=pltpu.CompilerParams(
            needs_layout_passes=False, use_tc_tiling_on_sc=False),
        scratch_types=[
            pltpu.VMEM((b_per_w,), jnp.int32),
            pltpu.VMEM((b_per_w,), jnp.int32),
            pltpu.VMEM((F, b_per_w), jnp.int32),
            pltpu.VMEM((F, b_per_w), jnp.int32),
            pltpu.VMEM((F, b_per_w), jnp.float32),
            pltpu.VMEM((F, b_per_w), jnp.float32),
            pltpu.VMEM((b_per_w,), jnp.float32),
            pltpu.VMEM((b_per_w,), jnp.float32),
            pltpu.VMEM((b_per_w,), jnp.float32),
            pltpu.SemaphoreType.DMA,
        ],
    )
    def gather_dot(u_idx_hbm, i_idx_hbm, cu_hbm, ci_hbm, ubt_hbm, ibt_hbm,
                   out_hbm, uidx_v, iidx_v, uwidx, iwidx, ubuf, ibuf,
                   ubv, ibv, out_v, sem):
        wid = lax.axis_index("s") * NC + lax.axis_index("c")
        base = wid * b_per_w
        pltpu.sync_copy(u_idx_hbm.at[pl.ds(base, b_per_w)], uidx_v)
        pltpu.sync_copy(i_idx_hbm.at[pl.ds(base, b_per_w)], iidx_v)

        def addrs(g, carry):
            for idx_v, widx in ((uidx_v, uwidx), (iidx_v, iwidx)):
                j = idx_v[pl.ds(g * L, L)]
                jm = j + (j >> 7) * (TS * TL - TL)
                for f in range(F):
                    off = (f // TS) * n_tc * TS * TL + (f % TS) * TL
                    widx[f, pl.ds(g * L, L)] = jm + off
            return carry

        lax.fori_loop(0, b_per_w // L, addrs, 0)

        handles = []
        for f in range(F):
            handles.append(pltpu.async_copy(cu_hbm.at[uwidx.at[f]],
                                            ubuf.at[f], sem))
            handles.append(pltpu.async_copy(ci_hbm.at[iwidx.at[f]],
                                            ibuf.at[f], sem))
        handles.append(pltpu.async_copy(ubt_hbm.at[0].at[uidx_v], ubv, sem))
        handles.append(pltpu.async_copy(ibt_hbm.at[0].at[iidx_v], ibv, sem))
        for h in handles:
            h.wait()

        def body(g, carry):
            acc = ubv[pl.ds(g * L, L)] + ibv[pl.ds(g * L, L)]
            for f in range(F):
                acc = acc + (ubuf[f, pl.ds(g * L, L)]
                             * ibuf[f, pl.ds(g * L, L)])
            out_v[pl.ds(g * L, L)] = acc
            return carry

        lax.fori_loop(0, b_per_w // L, body, 0)
        pltpu.sync_copy(out_v, out_hbm.at[pl.ds(base, b_per_w)])

    n_full = (n_tc - 1) * TL
    utp = jnp.pad(u_emb[n_full:], ((0, TL - (N - n_full)), (0, 0))).T
    itp = jnp.pad(i_emb[n_full:], ((0, TL - (N - n_full)), (0, 0))).T
    cu, ci = detile(u_emb.T, i_emb.T, utp, itp)
    return gather_dot(u_idx, i_idx, cu.reshape(flat_len), ci.reshape(flat_len),
                      u_bias.T, i_bias.T)


# speedup vs baseline: 1.0005x; 1.0005x over previous
import functools

import jax
import jax.numpy as jnp
from jax import lax
from jax.experimental import pallas as pl
from jax.experimental.pallas import tpu as pltpu
from jax.experimental.pallas import tpu_sc as plsc

L = 16
TS, TL = 8, 128
DEPTH = 8


def kernel(u_idx, i_idx, u_emb, i_emb, u_bias, i_bias):
    B = u_idx.shape[0]
    N, F = u_emb.shape
    info = plsc.get_sparse_core_info()
    NC, NS = info.num_cores, info.num_subcores
    NW = NC * NS
    b_per_w = B // NW

    n_tc = (N + TL - 1) // TL
    n_tiles = (F // TS) * n_tc
    tpw = (n_tiles + NW - 1) // NW
    flat_len = n_tiles * TS * TL

    mesh = plsc.VectorSubcoreMesh(core_axis_name="c", subcore_axis_name="s")

    @functools.partial(
        pl.kernel,
        mesh=mesh,
        out_type=(jax.ShapeDtypeStruct((n_tiles * TS, TL), jnp.float32),
                  jax.ShapeDtypeStruct((n_tiles * TS, TL), jnp.float32)),
        compiler_params=pltpu.CompilerParams(use_tc_tiling_on_sc=True),
        scratch_types=[pltpu.SemaphoreType.DMA],
    )
    def detile(ut_hbm, it_hbm, utp_hbm, itp_hbm, cu_hbm, ci_hbm, sem):
        wid = lax.axis_index("s") * NC + lax.axis_index("c")
        t0 = wid * tpw

        def fire(t, a, c, src_full, src_tail, dst):
            a8 = pl.multiple_of(a * TS, TS)
            d8 = pl.multiple_of(t * TS, TS)

            @pl.when(c != n_tc - 1)
            def _():
                cl = pl.multiple_of(c * TL, TL)
                pltpu.async_copy(src_full.at[pl.ds(a8, TS), pl.ds(cl, TL)],
                                 dst.at[pl.ds(d8, TS)], sem)

            @pl.when(c == n_tc - 1)
            def _():
                pltpu.async_copy(src_tail.at[pl.ds(a8, TS), :],
                                 dst.at[pl.ds(d8, TS)], sem)

        def drain(t, dst):
            d8 = pl.multiple_of(t * TS, TS)
            pltpu.make_async_copy(ut_hbm.at[pl.ds(0, TS), pl.ds(0, TL)],
                                  dst.at[pl.ds(d8, TS)], sem).wait()

        def body(k, carry):
            a, c = carry
            t = t0 + k

            @pl.when(t < n_tiles)
            def _():
                fire(t, a, c, ut_hbm, utp_hbm, cu_hbm)
                fire(t, a, c, it_hbm, itp_hbm, ci_hbm)

            t8 = t - DEPTH

            @pl.when((k >= DEPTH) & (t8 < n_tiles))
            def _():
                drain(t8, cu_hbm)
                drain(t8, ci_hbm)

            wrap = c == n_tc - 1
            return (jnp.where(wrap, a + 1, a), jnp.where(wrap, 0, c + 1))

        lax.fori_loop(0, tpw, body, (t0 // n_tc, t0 % n_tc))

        def tail(k, carry):
            t8 = t0 + tpw - DEPTH + k

            @pl.when(t8 < n_tiles)
            def _():
                drain(t8, cu_hbm)
                drain(t8, ci_hbm)

            return carry

        lax.fori_loop(0, DEPTH, tail, 0)

    @functools.partial(
        pl.kernel,
        mesh=mesh,
        out_type=jax.ShapeDtypeStruct((B,), jnp.float32),
        compiler_params=pltpu.CompilerParams(
            needs_layout_passes=False, use_tc_tiling_on_sc=False),
        scratch_types=[
            pltpu.VMEM((b_per_w,), jnp.int32),
            pltpu.VMEM((b_per_w,), jnp.int32),
            pltpu.VMEM((F, b_per_w), jnp.int32),
            pltpu.VMEM((F, b_per_w), jnp.int32),
            pltpu.VMEM((F, b_per_w), jnp.float32),
            pltpu.VMEM((F, b_per_w), jnp.float32),
            pltpu.VMEM((b_per_w,), jnp.float32),
            pltpu.VMEM((b_per_w,), jnp.float32),
            pltpu.VMEM((b_per_w,), jnp.float32),
            pltpu.SemaphoreType.DMA,
        ],
    )
    def gather_dot(u_idx_hbm, i_idx_hbm, cu_hbm, ci_hbm, ubt_hbm, ibt_hbm,
                   out_hbm, uidx_v, iidx_v, uwidx, iwidx, ubuf, ibuf,
                   ubv, ibv, out_v, sem):
        wid = lax.axis_index("s") * NC + lax.axis_index("c")
        base = wid * b_per_w
        pltpu.sync_copy(u_idx_hbm.at[pl.ds(base, b_per_w)], uidx_v)
        pltpu.sync_copy(i_idx_hbm.at[pl.ds(base, b_per_w)], iidx_v)

        def addrs(g, carry):
            for idx_v, widx in ((uidx_v, uwidx), (iidx_v, iwidx)):
                j = idx_v[pl.ds(g * L, L)]
                jm = j + (j >> 7) * (TS * TL - TL)
                for f in range(F):
                    off = (f // TS) * n_tc * TS * TL + (f % TS) * TL
                    widx[f, pl.ds(g * L, L)] = jm + off
            return carry

        lax.fori_loop(0, b_per_w // L, addrs, 0)

        handles = []
        for f in range(F):
            handles.append(pltpu.async_copy(cu_hbm.at[uwidx.at[f]],
                                            ubuf.at[f], sem))
            handles.append(pltpu.async_copy(ci_hbm.at[iwidx.at[f]],
                                            ibuf.at[f], sem))
        handles.append(pltpu.async_copy(ubt_hbm.at[0].at[uidx_v], ubv, sem))
        handles.append(pltpu.async_copy(ibt_hbm.at[0].at[iidx_v], ibv, sem))
        for h in handles:
            h.wait()

        def body(g, carry):
            acc = ubv[pl.ds(g * L, L)] + ibv[pl.ds(g * L, L)]
            for f in range(F):
                acc = acc + (ubuf[f, pl.ds(g * L, L)]
                             * ibuf[f, pl.ds(g * L, L)])
            out_v[pl.ds(g * L, L)] = acc
            return carry

        lax.fori_loop(0, b_per_w // L, body, 0)
        pltpu.sync_copy(out_v, out_hbm.at[pl.ds(base, b_per_w)])

    n_full = (n_tc - 1) * TL
    utp = jnp.pad(u_emb[n_full:], ((0, TL - (N - n_full)), (0, 0))).T
    itp = jnp.pad(i_emb[n_full:], ((0, TL - (N - n_full)), (0, 0))).T
    cu, ci = detile(u_emb.T, i_emb.T, utp, itp)
    return gather_dot(u_idx, i_idx, cu.reshape(flat_len), ci.reshape(flat_len),
                      u_bias.T, i_bias.T)


# speedup vs baseline: 28.1786x; 28.1634x over previous
import functools

import jax
import jax.numpy as jnp
from jax import lax
from jax.experimental import pallas as pl
from jax.experimental.pallas import tpu as pltpu
from jax.experimental.pallas import tpu_sc as plsc

L = 16
TS, TL = 8, 128
DEPTH = 8


def kernel(u_idx, i_idx, u_emb, i_emb, u_bias, i_bias):
    B = u_idx.shape[0]
    N, F = u_emb.shape
    info = plsc.get_sparse_core_info()
    NC, NS = info.num_cores, info.num_subcores
    NW = NC * NS
    b_per_w = B // NW

    n_tc = (N + TL - 1) // TL
    n_tiles = (F // TS) * n_tc
    tpw = (n_tiles + NW - 1) // NW
    flat_len = n_tiles * TS * TL

    mesh = plsc.VectorSubcoreMesh(core_axis_name="c", subcore_axis_name="s")

    @functools.partial(
        pl.kernel,
        mesh=mesh,
        out_type=(jax.ShapeDtypeStruct((n_tiles * TS, TL), jnp.float32),
                  jax.ShapeDtypeStruct((n_tiles * TS, TL), jnp.float32)),
        compiler_params=pltpu.CompilerParams(use_tc_tiling_on_sc=True),
        scratch_types=[
            pltpu.VMEM((DEPTH, TS, TL), jnp.float32),
            pltpu.VMEM((DEPTH, TS, TL), jnp.float32),
            pltpu.SemaphoreType.DMA,
            pltpu.SemaphoreType.DMA,
            pltpu.SemaphoreType.DMA,
            pltpu.SemaphoreType.DMA,
        ],
    )
    def detile(ut_hbm, it_hbm, utp_hbm, itp_hbm, cu_hbm, ci_hbm,
               ubnc, ibnc, sin_u, sin_i, sout_u, sout_i):
        wid = lax.axis_index("s") * NC + lax.axis_index("c")
        t0 = wid * tpw
        PD = 4

        def fire_in(a, c, s, src_full, src_tail, bnc, sem):
            a8 = pl.multiple_of(a * TS, TS)

            @pl.when(c != n_tc - 1)
            def _():
                cl = pl.multiple_of(c * TL, TL)
                pltpu.async_copy(src_full.at[pl.ds(a8, TS), pl.ds(cl, TL)],
                                 bnc.at[s], sem)

            @pl.when(c == n_tc - 1)
            def _():
                pltpu.async_copy(src_tail.at[pl.ds(a8, TS), :],
                                 bnc.at[s], sem)

        def wait_one(bnc, s, sem):
            pltpu.make_async_copy(ut_hbm.at[pl.ds(0, TS), pl.ds(0, TL)],
                                  bnc.at[s], sem).wait()

        def fire_out(t, s, bnc, dst, sem):
            d8 = pl.multiple_of(t * TS, TS)
            pltpu.async_copy(bnc.at[s], dst.at[pl.ds(d8, TS)], sem)

        def body(k, carry):
            a, c = carry
            t = t0 + k
            s = k & (DEPTH - 1)

            @pl.when((k >= DEPTH) & (t - DEPTH < n_tiles))
            def _():
                wait_one(ubnc, s, sout_u)
                wait_one(ibnc, s, sout_i)

            @pl.when(t < n_tiles)
            def _():
                fire_in(a, c, s, ut_hbm, utp_hbm, ubnc, sin_u)
                fire_in(a, c, s, it_hbm, itp_hbm, ibnc, sin_i)

            kp = k - PD
            tp = t0 + kp
            sp = kp & (DEPTH - 1)

            @pl.when((k >= PD) & (tp < n_tiles))
            def _():
                wait_one(ubnc, sp, sin_u)
                wait_one(ibnc, sp, sin_i)
                fire_out(tp, sp, ubnc, cu_hbm, sout_u)
                fire_out(tp, sp, ibnc, ci_hbm, sout_i)

            wrap = c == n_tc - 1
            return (jnp.where(wrap, a + 1, a), jnp.where(wrap, 0, c + 1))

        lax.fori_loop(0, tpw, body, (t0 // n_tc, t0 % n_tc))

        def tail(j, carry):
            kp = tpw - PD + j
            tp = t0 + kp
            sp = kp & (DEPTH - 1)

            @pl.when(tp < n_tiles)
            def _():
                wait_one(ubnc, sp, sin_u)
                wait_one(ibnc, sp, sin_i)
                fire_out(tp, sp, ubnc, cu_hbm, sout_u)
                fire_out(tp, sp, ibnc, ci_hbm, sout_i)

            return carry

        lax.fori_loop(0, PD, tail, 0)

        def flush(j, carry):
            t8 = t0 + tpw - DEPTH + j

            @pl.when(t8 < n_tiles)
            def _():
                wait_one(ubnc, j, sout_u)
                wait_one(ibnc, j, sout_i)

            return carry

        lax.fori_loop(0, DEPTH, flush, 0)

    @functools.partial(
        pl.kernel,
        mesh=mesh,
        out_type=jax.ShapeDtypeStruct((B,), jnp.float32),
        compiler_params=pltpu.CompilerParams(
            needs_layout_passes=False, use_tc_tiling_on_sc=False),
        scratch_types=[
            pltpu.VMEM((b_per_w,), jnp.int32),
            pltpu.VMEM((b_per_w,), jnp.int32),
            pltpu.VMEM((F, b_per_w), jnp.int32),
            pltpu.VMEM((F, b_per_w), jnp.int32),
            pltpu.VMEM((F, b_per_w), jnp.float32),
            pltpu.VMEM((F, b_per_w), jnp.float32),
            pltpu.VMEM((b_per_w,), jnp.float32),
            pltpu.VMEM((b_per_w,), jnp.float32),
            pltpu.VMEM((b_per_w,), jnp.float32),
            pltpu.SemaphoreType.DMA,
        ],
    )
    def gather_dot(u_idx_hbm, i_idx_hbm, cu_hbm, ci_hbm, ubt_hbm, ibt_hbm,
                   out_hbm, uidx_v, iidx_v, uwidx, iwidx, ubuf, ibuf,
                   ubv, ibv, out_v, sem):
        wid = lax.axis_index("s") * NC + lax.axis_index("c")
        base = wid * b_per_w
        pltpu.sync_copy(u_idx_hbm.at[pl.ds(base, b_per_w)], uidx_v)
        pltpu.sync_copy(i_idx_hbm.at[pl.ds(base, b_per_w)], iidx_v)

        def addrs(g, carry):
            for idx_v, widx in ((uidx_v, uwidx), (iidx_v, iwidx)):
                j = idx_v[pl.ds(g * L, L)]
                jm = j + (j >> 7) * (TS * TL - TL)
                for f in range(F):
                    off = (f // TS) * n_tc * TS * TL + (f % TS) * TL
                    widx[f, pl.ds(g * L, L)] = jm + off
            return carry

        lax.fori_loop(0, b_per_w // L, addrs, 0)

        handles = []
        for f in range(F):
            handles.append(pltpu.async_copy(cu_hbm.at[uwidx.at[f]],
                                            ubuf.at[f], sem))
            handles.append(pltpu.async_copy(ci_hbm.at[iwidx.at[f]],
                                            ibuf.at[f], sem))
        handles.append(pltpu.async_copy(ubt_hbm.at[0].at[uidx_v], ubv, sem))
        handles.append(pltpu.async_copy(ibt_hbm.at[0].at[iidx_v], ibv, sem))
        for h in handles:
            h.wait()

        def body(g, carry):
            acc = ubv[pl.ds(g * L, L)] + ibv[pl.ds(g * L, L)]
            for f in range(F):
                acc = acc + (ubuf[f, pl.ds(g * L, L)]
                             * ibuf[f, pl.ds(g * L, L)])
            out_v[pl.ds(g * L, L)] = acc
            return carry

        lax.fori_loop(0, b_per_w // L, body, 0)
        pltpu.sync_copy(out_v, out_hbm.at[pl.ds(base, b_per_w)])

    n_full = (n_tc - 1) * TL
    utp = jnp.pad(u_emb[n_full:], ((0, TL - (N - n_full)), (0, 0))).T
    itp = jnp.pad(i_emb[n_full:], ((0, TL - (N - n_full)), (0, 0))).T
    cu, ci = detile(u_emb.T, i_emb.T, utp, itp)
    return gather_dot(u_idx, i_idx, cu.reshape(flat_len), ci.reshape(flat_len),
                      u_bias.T, i_bias.T)


# speedup vs baseline: 31.6693x; 1.1239x over previous
import functools

import jax
import jax.numpy as jnp
from jax import lax
from jax.experimental import pallas as pl
from jax.experimental.pallas import tpu as pltpu
from jax.experimental.pallas import tpu_sc as plsc

L = 16
TS, TL = 8, 128
DEPTH = 4
GW = 8


def kernel(u_idx, i_idx, u_emb, i_emb, u_bias, i_bias):
    B = u_idx.shape[0]
    N, F = u_emb.shape
    info = plsc.get_sparse_core_info()
    NC, NS = info.num_cores, info.num_subcores
    NW = NC * NS
    b_per_w = B // NW

    n_tc = (N + TL - 1) // TL
    n_tiles = (F // TS) * n_tc
    tpw = (n_tiles + NW - 1) // NW
    flat_len = n_tiles * TS * TL

    mesh = plsc.VectorSubcoreMesh(core_axis_name="c", subcore_axis_name="s")

    @functools.partial(
        pl.kernel,
        mesh=mesh,
        out_type=(jax.ShapeDtypeStruct((n_tiles * TS, TL), jnp.float32),
                  jax.ShapeDtypeStruct((n_tiles * TS, TL), jnp.float32)),
        compiler_params=pltpu.CompilerParams(use_tc_tiling_on_sc=True),
        scratch_types=[
            pltpu.VMEM((DEPTH, TS, GW * TL), jnp.float32),
            pltpu.VMEM((DEPTH, TS, GW * TL), jnp.float32),
            pltpu.SemaphoreType.DMA,
            pltpu.SemaphoreType.DMA,
            pltpu.SemaphoreType.DMA,
            pltpu.SemaphoreType.DMA,
        ],
    )
    def detile(ut_hbm, it_hbm, utp_hbm, itp_hbm, cu_hbm, ci_hbm,
               ubnc, ibnc, sin_u, sin_i, sout_u, sout_i):
        wid = lax.axis_index("s") * NC + lax.axis_index("c")
        PD = 2
        n_grp = (n_tc - 1) // GW
        gpw = n_grp * (F // TS) // NW
        g0 = wid * gpw

        def wait_grp(bnc, s, sem):
            pltpu.make_async_copy(ut_hbm.at[pl.ds(0, TS), pl.ds(0, GW * TL)],
                                  bnc.at[s], sem).wait()

        def body(k, carry):
            a, j = carry
            s = k & (DEPTH - 1)

            @pl.when(k >= DEPTH)
            def _():
                wait_grp(ubnc, s, sout_u)
                wait_grp(ibnc, s, sout_i)

            a8 = pl.multiple_of(a * TS, TS)
            cl = pl.multiple_of(j * GW * TL, TL)
            pltpu.async_copy(ut_hbm.at[pl.ds(a8, TS), pl.ds(cl, GW * TL)],
                             ubnc.at[s], sin_u)
            pltpu.async_copy(it_hbm.at[pl.ds(a8, TS), pl.ds(cl, GW * TL)],
                             ibnc.at[s], sin_i)

            kp = k - PD
            sp = kp & (DEPTH - 1)
            ap, jp = carry_back(a, j, kp)

            @pl.when(k >= PD)
            def _():
                wait_grp(ubnc, sp, sin_u)
                wait_grp(ibnc, sp, sin_i)
                t_base = ap * n_tc + jp * GW
                for w in range(GW):
                    d8 = pl.multiple_of((t_base + w) * TS, TS)
                    pltpu.async_copy(ubnc.at[sp].at[:, pl.ds(w * TL, TL)],
                                     cu_hbm.at[pl.ds(d8, TS)], sout_u)
                    pltpu.async_copy(ibnc.at[sp].at[:, pl.ds(w * TL, TL)],
                                     ci_hbm.at[pl.ds(d8, TS)], sout_i)

            wrap = j == n_grp - 1
            return (jnp.where(wrap, a + 1, a), jnp.where(wrap, 0, j + 1))

        def carry_back(a, j, kp):
            jb = j - PD
            under = jb < 0
            return (jnp.where(under, a - 1, a),
                    jnp.where(under, jb + n_grp, jb))

        lax.fori_loop(0, gpw, body, (g0 // n_grp, g0 % n_grp))

        def tail(q, carry):
            kp = gpw - PD + q
            gp = g0 + kp
            sp = kp & (DEPTH - 1)
            ap = gp // n_grp
            jp = gp % n_grp
            wait_grp(ubnc, sp, sin_u)
            wait_grp(ibnc, sp, sin_i)
            t_base = ap * n_tc + jp * GW
            for w in range(GW):
                d8 = pl.multiple_of((t_base + w) * TS, TS)
                pltpu.async_copy(ubnc.at[sp].at[:, pl.ds(w * TL, TL)],
                                 cu_hbm.at[pl.ds(d8, TS)], sout_u)
                pltpu.async_copy(ibnc.at[sp].at[:, pl.ds(w * TL, TL)],
                                 ci_hbm.at[pl.ds(d8, TS)], sout_i)
            return carry

        lax.fori_loop(0, PD, tail, 0)

        def flush(q, carry):
            wait_grp(ubnc, q & (DEPTH - 1), sout_u)
            wait_grp(ibnc, q & (DEPTH - 1), sout_i)
            return carry

        lax.fori_loop(0, min(DEPTH, gpw), flush, 0)

        n_rem_c = n_tc - n_grp * GW
        n_rem = 2 * (F // TS) * n_rem_c

        def rem_item(r):
            tbl = r % 2
            rest = r // 2
            ar = rest // n_rem_c
            cr = n_grp * GW + rest % n_rem_c
            src_full, src_tail = (ut_hbm, utp_hbm) if tbl == 0 else (it_hbm, itp_hbm)
            dst = cu_hbm if tbl == 0 else ci_hbm
            a8 = pl.multiple_of(ar * TS, TS)
            d8 = pl.multiple_of((ar * n_tc + cr) * TS, TS)
            piece = ubnc.at[0].at[:, pl.ds(0, TL)]
            if cr == n_tc - 1:
                pltpu.async_copy(src_tail.at[pl.ds(a8, TS), :], piece, sin_u)
            else:
                cl = pl.multiple_of(cr * TL, TL)
                pltpu.async_copy(src_full.at[pl.ds(a8, TS), pl.ds(cl, TL)],
                                 piece, sin_u)
            pltpu.make_async_copy(ut_hbm.at[pl.ds(0, TS), pl.ds(0, TL)],
                                  piece, sin_u).wait()
            pltpu.sync_copy(piece, dst.at[pl.ds(d8, TS)])

        for r0 in range(0, n_rem, NW):
            for r in range(r0, min(r0 + NW, n_rem)):
                @pl.when(wid == (r - r0))
                def _(r=r):
                    rem_item(r)

    @functools.partial(
        pl.kernel,
        mesh=mesh,
        out_type=jax.ShapeDtypeStruct((B,), jnp.float32),
        compiler_params=pltpu.CompilerParams(
            needs_layout_passes=False, use_tc_tiling_on_sc=False),
        scratch_types=[
            pltpu.VMEM((b_per_w,), jnp.int32),
            pltpu.VMEM((b_per_w,), jnp.int32),
            pltpu.VMEM((F, b_per_w), jnp.int32),
            pltpu.VMEM((F, b_per_w), jnp.int32),
            pltpu.VMEM((F, b_per_w), jnp.float32),
            pltpu.VMEM((F, b_per_w), jnp.float32),
            pltpu.VMEM((b_per_w,), jnp.float32),
            pltpu.VMEM((b_per_w,), jnp.float32),
            pltpu.VMEM((b_per_w,), jnp.float32),
            pltpu.SemaphoreType.DMA,
        ],
    )
    def gather_dot(u_idx_hbm, i_idx_hbm, cu_hbm, ci_hbm, ubt_hbm, ibt_hbm,
                   out_hbm, uidx_v, iidx_v, uwidx, iwidx, ubuf, ibuf,
                   ubv, ibv, out_v, sem):
        wid = lax.axis_index("s") * NC + lax.axis_index("c")
        base = wid * b_per_w
        pltpu.sync_copy(u_idx_hbm.at[pl.ds(base, b_per_w)], uidx_v)
        pltpu.sync_copy(i_idx_hbm.at[pl.ds(base, b_per_w)], iidx_v)

        def addrs(g, carry):
            for idx_v, widx in ((uidx_v, uwidx), (iidx_v, iwidx)):
                j = idx_v[pl.ds(g * L, L)]
                jm = j + (j >> 7) * (TS * TL - TL)
                for f in range(F):
                    off = (f // TS) * n_tc * TS * TL + (f % TS) * TL
                    widx[f, pl.ds(g * L, L)] = jm + off
            return carry

        lax.fori_loop(0, b_per_w // L, addrs, 0)

        handles = []
        for f in range(F):
            handles.append(pltpu.async_copy(cu_hbm.at[uwidx.at[f]],
                                            ubuf.at[f], sem))
            handles.append(pltpu.async_copy(ci_hbm.at[iwidx.at[f]],
                                            ibuf.at[f], sem))
        handles.append(pltpu.async_copy(ubt_hbm.at[0].at[uidx_v], ubv, sem))
        handles.append(pltpu.async_copy(ibt_hbm.at[0].at[iidx_v], ibv, sem))
        for h in handles:
            h.wait()

        def body(g, carry):
            acc = ubv[pl.ds(g * L, L)] + ibv[pl.ds(g * L, L)]
            for f in range(F):
                acc = acc + (ubuf[f, pl.ds(g * L, L)]
                             * ibuf[f, pl.ds(g * L, L)])
            out_v[pl.ds(g * L, L)] = acc
            return carry

        lax.fori_loop(0, b_per_w // L, body, 0)
        pltpu.sync_copy(out_v, out_hbm.at[pl.ds(base, b_per_w)])

    n_full = (n_tc - 1) * TL
    utp = jnp.pad(u_emb[n_full:], ((0, TL - (N - n_full)), (0, 0))).T
    itp = jnp.pad(i_emb[n_full:], ((0, TL - (N - n_full)), (0, 0))).T
    cu, ci = detile(u_emb.T, i_emb.T, utp, itp)
    return gather_dot(u_idx, i_idx, cu.reshape(flat_len), ci.reshape(flat_len),
                      u_bias.T, i_bias.T)
